# baseline (speedup 1.0000x reference)
import functools

import jax
import numpy as np
import jax.numpy as jnp
from jax import lax
from jax.experimental import pallas as pl
from jax.experimental.pallas import tpu as pltpu
from jax.experimental.pallas import tpu_sc as plsc

N = 10000
E = 320000
D = 128
G = 64
NC = 2
NS = 16
NW = NC * NS
CHUNK = 64
NCHUNK = 160
SB = 4
NSUPER = NCHUNK // SB
EP = NW * NCHUNK * CHUNK
NPAD = 10240
RB = 2000
NRB = N // RB

_mesh = plsc.VectorSubcoreMesh(core_axis_name="c", subcore_axis_name="s")

_PAD_SRC = ((np.arange(EP - E) * 131) % N).astype(np.int32)
_PAD_DST = (N + np.arange(EP - E) % (NPAD - N)).astype(np.int32)


@functools.partial(
    pl.kernel,
    out_type=jax.ShapeDtypeStruct((NC, NPAD), jnp.float32),
    mesh=_mesh,
    scratch_types=[
        pltpu.VMEM((SB, CHUNK), jnp.int32),
        pltpu.VMEM((CHUNK,), jnp.float32),
        pltpu.VMEM((NPAD // NS,), jnp.float32),
        pltpu.VMEM_SHARED((NPAD,), jnp.float32),
    ],
)
def _sc_degree(dst4, ones_hbm, zeros_hbm, out, idxb, onev, stage, degs):
    cid = lax.axis_index("c")
    sid = lax.axis_index("s")
    wid = cid * NS + sid
    seg = NPAD // NS

    pltpu.sync_copy(zeros_hbm, stage)
    pltpu.sync_copy(stage, degs.at[pl.ds(sid * seg, seg)])
    pltpu.sync_copy(ones_hbm, onev)
    plsc.subcore_barrier()

    def body(s, _):
        pltpu.sync_copy(dst4.at[wid, s], idxb)
        for k in range(SB):
            pltpu.sync_copy(onev, degs.at[idxb.at[k]], add=True)
        return 0

    lax.fori_loop(0, NSUPER, body, 0)
    plsc.subcore_barrier()

    pltpu.sync_copy(degs.at[pl.ds(sid * seg, seg)], stage)
    pltpu.sync_copy(stage, out.at[cid, pl.ds(sid * seg, seg)])


@functools.partial(
    pl.kernel,
    out_type=jax.ShapeDtypeStruct((NC, NPAD, D), jnp.float32),
    mesh=_mesh,
    scratch_types=[
        pltpu.VMEM((2 * SB, CHUNK), jnp.int32),
        pltpu.VMEM((2 * SB, CHUNK), jnp.int32),
        pltpu.VMEM((CHUNK, D), jnp.float32),
        pltpu.VMEM((CHUNK, D), jnp.float32),
        pltpu.VMEM((CHUNK, D), jnp.float32),
        pltpu.VMEM((CHUNK, D), jnp.float32),
        pltpu.VMEM_SHARED((NPAD, D), jnp.float32),
        pltpu.SemaphoreType.DMA,
        pltpu.SemaphoreType.DMA,
        pltpu.SemaphoreType.DMA,
        pltpu.SemaphoreType.DMA,
        pltpu.SemaphoreType.DMA,
        pltpu.SemaphoreType.DMA,
    ],
)
def _sc_aggregate(h, src4, dst4, zrows_hbm, out, idxa, idxb, b0, b1, b2, b3,
                  acc, semia, semib, g0, g1, g2, g3):
    cid = lax.axis_index("c")
    sid = lax.axis_index("s")
    wid = cid * NS + sid
    rows_per_tile = NPAD // NS
    nzcopy = rows_per_tile // CHUNK

    gbufs = (b0, b1, b2, b3)
    gsems = (g0, g1, g2, g3)

    def gstart(iref, row, slot):
        pltpu.async_copy(h.at[iref.at[row]], gbufs[slot], gsems[slot])

    def gwait(iref, row, slot):
        pltpu.make_async_copy(h.at[iref.at[row]], gbufs[slot],
                              gsems[slot]).wait()

    def istart(iref, sup, sem):
        pltpu.async_copy(src4.at[wid, sup], iref.at[pl.ds(0, SB)], sem)
        pltpu.async_copy(dst4.at[wid, sup], iref.at[pl.ds(SB, SB)], sem)

    def iwait(iref, sup, sem):
        pltpu.make_async_copy(src4.at[wid, sup],
                              iref.at[pl.ds(0, SB)], sem).wait()
        pltpu.make_async_copy(dst4.at[wid, sup],
                              iref.at[pl.ds(SB, SB)], sem).wait()

    pltpu.sync_copy(zrows_hbm, b0)
    for k in range(nzcopy):
        pltpu.sync_copy(b0, acc.at[pl.ds(sid * rows_per_tile + k * CHUNK,
                                         CHUNK)])
    plsc.subcore_barrier()

    def process_super(icur, inxt, snxt, inxt_sem):
        for k in range(SB):
            if k == 0:
                gstart(icur, 3, 3)
            elif k == 1:
                iwait(inxt, snxt, inxt_sem)
                gstart(inxt, 0, 0)
            else:
                gstart(inxt, k - 1, k - 1)
            gwait(icur, k, k)
            pltpu.sync_copy(gbufs[k], acc.at[icur.at[SB + k]], add=True)

    pltpu.sync_copy(src4.at[wid, 0], idxa.at[pl.ds(0, SB)])
    pltpu.sync_copy(dst4.at[wid, 0], idxa.at[pl.ds(SB, SB)])
    for k in range(3):
        gstart(idxa, k, k)
    istart(idxb, 1, semib)

    def pair(t, _):
        s0 = 2 * t
        process_super(idxa, idxb, s0 + 1, semib)
        s2 = jnp.minimum(s0 + 2, NSUPER - 1)
        istart(idxa, s2, semia)
        process_super(idxb, idxa, s2, semia)
        s3 = jnp.minimum(s0 + 3, NSUPER - 1)
        istart(idxb, s3, semib)
        return 0

    lax.fori_loop(0, NSUPER // 2, pair, 0)
    for k in range(3):
        gwait(idxa, k, k)
    iwait(idxb, NSUPER - 1, semib)
    plsc.subcore_barrier()

    for k in range(nzcopy):
        r0 = sid * rows_per_tile + k * CHUNK
        pltpu.sync_copy(acc.at[pl.ds(r0, CHUNK)], b0)
        pltpu.sync_copy(b0, out.at[cid, pl.ds(r0, CHUNK)])


def _tc_mm_body(x_ref, w_ref, h_ref):
    h_ref[...] = jnp.dot(x_ref[...], w_ref[...],
                         preferred_element_type=jnp.float32,
                         precision=lax.Precision.HIGHEST)


def _tc_scale_body(h_ref, d0_ref, d1_ref, hp_ref, dinv_ref):
    deg = d0_ref[...] + d1_ref[...] + 1.0
    dinv = lax.rsqrt(deg)
    hp_ref[...] = h_ref[...] * dinv
    dinv_ref[...] = dinv


def _tc_mid_body(acc0_ref, acc1_ref, hp_ref, dinv_ref, b_ref, w_ref, out_ref):
    dinv = dinv_ref[...]
    z = (acc0_ref[0] + acc1_ref[0] + hp_ref[...]) * dinv + b_ref[...]
    z = jnp.maximum(z, 0.0)
    out_ref[...] = jnp.dot(
        z, w_ref[...], preferred_element_type=jnp.float32,
        precision=lax.Precision.HIGHEST) * dinv


def _tc_last_body(acc0_ref, acc1_ref, hp_ref, dinv_ref, b_ref, batch_ref,
                  fcw_ref, fcb_ref, out_ref, sums_ref, cnts_ref):
    i = pl.program_id(0)
    z = (acc0_ref[0] + acc1_ref[0] + hp_ref[...]) * dinv_ref[...] + b_ref[...]
    z = jnp.maximum(z, 0.0)
    gid = lax.broadcasted_iota(jnp.int32, (RB, G), 1)
    onehot = (batch_ref[...] == gid).astype(jnp.float32)
    ps = lax.dot_general(onehot, z, (((0,), (0,)), ((), ())),
                         preferred_element_type=jnp.float32,
        precision=lax.Precision.HIGHEST)
    pc = lax.dot_general(onehot, jnp.ones_like(z), (((0,), (0,)), ((), ())),
                         preferred_element_type=jnp.float32,
        precision=lax.Precision.HIGHEST)

    @pl.when(i == 0)
    def _():
        sums_ref[...] = ps
        cnts_ref[...] = pc

    @pl.when(i > 0)
    def _():
        sums_ref[...] += ps
        cnts_ref[...] += pc

    @pl.when(i == NRB - 1)
    def _():
        g = sums_ref[...] / jnp.maximum(cnts_ref[...], 1.0)
        out_ref[...] = jnp.dot(
            g, fcw_ref[...], preferred_element_type=jnp.float32,
        precision=lax.Precision.HIGHEST) + fcb_ref[...]


_row_spec = pl.BlockSpec((RB, D), lambda i: (i, 0))
_col_spec = pl.BlockSpec((RB, 1), lambda i: (i, 0))
_full_w = pl.BlockSpec((D, D), lambda i: (0, 0))
_full_b = pl.BlockSpec((1, D), lambda i: (0, 0))
_acc0_spec = pl.BlockSpec((1, RB, D), lambda i: (0, i, 0))
_acc1_spec = pl.BlockSpec((1, RB, D), lambda i: (1, i, 0))

_tc_mm = pl.pallas_call(
    _tc_mm_body,
    grid=(NRB,),
    in_specs=[_row_spec, _full_w],
    out_specs=_row_spec,
    out_shape=jax.ShapeDtypeStruct((N, D), jnp.float32),
)

_tc_scale = pl.pallas_call(
    _tc_scale_body,
    grid=(NRB,),
    in_specs=[_row_spec, _col_spec, _col_spec],
    out_specs=[_row_spec, _col_spec],
    out_shape=[jax.ShapeDtypeStruct((N, D), jnp.float32),
               jax.ShapeDtypeStruct((N, 1), jnp.float32)],
)

_tc_mid = pl.pallas_call(
    _tc_mid_body,
    grid=(NRB,),
    in_specs=[_acc0_spec, _acc1_spec, _row_spec, _col_spec, _full_b, _full_w],
    out_specs=_row_spec,
    out_shape=jax.ShapeDtypeStruct((N, D), jnp.float32),
)

_tc_last = pl.pallas_call(
    _tc_last_body,
    grid=(NRB,),
    in_specs=[_acc0_spec, _acc1_spec, _row_spec, _col_spec, _full_b,
              pl.BlockSpec((RB, 1), lambda i: (i, 0)),
              pl.BlockSpec((D, 1), lambda i: (0, 0)),
              pl.BlockSpec((1, 1), lambda i: (0, 0))],
    out_specs=pl.BlockSpec((G, 1), lambda i: (0, 0)),
    out_shape=jax.ShapeDtypeStruct((G, 1), jnp.float32),
    scratch_shapes=[pltpu.VMEM((G, D), jnp.float32),
                    pltpu.VMEM((G, D), jnp.float32)],
)


def kernel(x, edge_index, batch, W1, b1, W2, b2, fcW, fcb):
    src = edge_index[0].astype(jnp.int32)
    dst = edge_index[1].astype(jnp.int32)
    src4 = jnp.concatenate([src, _PAD_SRC]).reshape(NW, NSUPER, SB, CHUNK)

    dst4 = jnp.concatenate([dst, _PAD_DST]).reshape(NW, NSUPER, SB, CHUNK)

    ones_h = jnp.ones((CHUNK,), jnp.float32)
    zeros1 = jnp.zeros((NPAD // NS,), jnp.float32)
    zrows = jnp.zeros((CHUNK, D), jnp.float32)

    deg_parts = _sc_degree(dst4, ones_h, zeros1)
    h1 = _tc_mm(x, W1)
    d0 = deg_parts[0, :N].reshape(N, 1)
    d1 = deg_parts[1, :N].reshape(N, 1)
    h1p, dinv = _tc_scale(h1, d0, d1)

    acc1 = _sc_aggregate(h1p, src4, dst4, zrows)
    h2p = _tc_mid(acc1, acc1, h1p, dinv, b1.reshape(1, D), W2)

    acc2 = _sc_aggregate(h2p, src4, dst4, zrows)
    out = _tc_last(acc2, acc2, h2p, dinv, b2.reshape(1, D),
                   batch.astype(jnp.int32).reshape(N, 1), fcW,
                   fcb.reshape(1, 1))
    return out


# speedup vs baseline: 1.0268x; 1.0268x over previous
import functools

import jax
import numpy as np
import jax.numpy as jnp
from jax import lax
from jax.experimental import pallas as pl
from jax.experimental.pallas import tpu as pltpu
from jax.experimental.pallas import tpu_sc as plsc

N = 10000
E = 320000
D = 128
G = 64
NC = 2
NS = 16
NW = NC * NS
CHUNK = 128
NCHUNK = 80
SB = 4
NSUPER = NCHUNK // SB
EP = NW * NCHUNK * CHUNK
NPAD = 10240
RB = 2000
NRB = N // RB

_mesh = plsc.VectorSubcoreMesh(core_axis_name="c", subcore_axis_name="s")

_PAD_SRC = ((np.arange(EP - E) * 131) % N).astype(np.int32)
_PAD_DST = (N + np.arange(EP - E) % (NPAD - N)).astype(np.int32)


@functools.partial(
    pl.kernel,
    out_type=jax.ShapeDtypeStruct((NC, NPAD), jnp.float32),
    mesh=_mesh,
    scratch_types=[
        pltpu.VMEM((SB, CHUNK), jnp.int32),
        pltpu.VMEM((CHUNK,), jnp.float32),
        pltpu.VMEM((NPAD // NS,), jnp.float32),
        pltpu.VMEM_SHARED((NPAD,), jnp.float32),
    ],
)
def _sc_degree(dst4, ones_hbm, zeros_hbm, out, idxb, onev, stage, degs):
    cid = lax.axis_index("c")
    sid = lax.axis_index("s")
    wid = cid * NS + sid
    seg = NPAD // NS

    pltpu.sync_copy(zeros_hbm, stage)
    pltpu.sync_copy(stage, degs.at[pl.ds(sid * seg, seg)])
    pltpu.sync_copy(ones_hbm, onev)
    plsc.subcore_barrier()

    def body(s, _):
        pltpu.sync_copy(dst4.at[wid, s], idxb)
        for k in range(SB):
            pltpu.sync_copy(onev, degs.at[idxb.at[k]], add=True)
        return 0

    lax.fori_loop(0, NSUPER, body, 0)
    plsc.subcore_barrier()

    pltpu.sync_copy(degs.at[pl.ds(sid * seg, seg)], stage)
    pltpu.sync_copy(stage, out.at[cid, pl.ds(sid * seg, seg)])


@functools.partial(
    pl.kernel,
    out_type=jax.ShapeDtypeStruct((NC, NPAD, D), jnp.float32),
    mesh=_mesh,
    scratch_types=[
        pltpu.VMEM((2 * SB, CHUNK), jnp.int32),
        pltpu.VMEM((2 * SB, CHUNK), jnp.int32),
        pltpu.VMEM((CHUNK, D), jnp.float32),
        pltpu.VMEM((CHUNK, D), jnp.float32),
        pltpu.VMEM_SHARED((NPAD, D), jnp.float32),
        pltpu.SemaphoreType.DMA,
        pltpu.SemaphoreType.DMA,
        pltpu.SemaphoreType.DMA,
        pltpu.SemaphoreType.DMA,
    ],
)
def _sc_aggregate(h, src4, dst4, zrows_hbm, out, idxa, idxb, bufa, bufb, acc,
                  semia, semib, sema, semb):
    cid = lax.axis_index("c")
    sid = lax.axis_index("s")
    wid = cid * NS + sid
    rows_per_tile = NPAD // NS
    nzcopy = rows_per_tile // CHUNK

    pltpu.sync_copy(zrows_hbm, bufa)
    for k in range(nzcopy):
        pltpu.async_copy(bufa, acc.at[pl.ds(sid * rows_per_tile + k * CHUNK,
                                            CHUNK)], semb)
    for k in range(nzcopy):
        pltpu.make_async_copy(bufa, acc.at[pl.ds(sid * rows_per_tile
                                                 + k * CHUNK, CHUNK)],
                              semb).wait()
    plsc.subcore_barrier()

    gbufs = (bufa, bufb)
    gsems = (sema, semb)

    def process_super(icur, inxt, snxt, inxt_sem):
        for k in range(SB):
            cur, csem = gbufs[k % 2], gsems[k % 2]
            nxt, nsem = gbufs[(k + 1) % 2], gsems[(k + 1) % 2]
            if k < SB - 1:
                pltpu.async_copy(h.at[icur.at[k + 1]], nxt, nsem)
            else:
                pltpu.make_async_copy(src4.at[wid, snxt],
                                      inxt.at[pl.ds(0, SB)], inxt_sem).wait()
                pltpu.make_async_copy(dst4.at[wid, snxt],
                                      inxt.at[pl.ds(SB, SB)], inxt_sem).wait()
                pltpu.async_copy(h.at[inxt.at[0]], nxt, nsem)
            pltpu.make_async_copy(h.at[icur.at[k]], cur, csem).wait()
            pltpu.sync_copy(cur, acc.at[icur.at[SB + k]], add=True)

    pltpu.sync_copy(src4.at[wid, 0], idxa.at[pl.ds(0, SB)])
    pltpu.sync_copy(dst4.at[wid, 0], idxa.at[pl.ds(SB, SB)])
    pltpu.async_copy(h.at[idxa.at[0]], bufa, sema)

    def pair(t, _):
        s0 = 2 * t
        pltpu.async_copy(src4.at[wid, s0 + 1], idxb.at[pl.ds(0, SB)], semib)
        pltpu.async_copy(dst4.at[wid, s0 + 1], idxb.at[pl.ds(SB, SB)], semib)
        process_super(idxa, idxb, s0 + 1, semib)
        s2 = jnp.minimum(s0 + 2, NSUPER - 1)
        pltpu.async_copy(src4.at[wid, s2], idxa.at[pl.ds(0, SB)], semia)
        pltpu.async_copy(dst4.at[wid, s2], idxa.at[pl.ds(SB, SB)], semia)
        process_super(idxb, idxa, s2, semia)
        return 0

    lax.fori_loop(0, NSUPER // 2, pair, 0)
    pltpu.make_async_copy(h.at[idxa.at[0]], bufa, sema).wait()
    plsc.subcore_barrier()

    obufs = (bufa, bufb)
    osems = (sema, semb)
    r00 = sid * rows_per_tile
    pltpu.async_copy(acc.at[pl.ds(r00, CHUNK)], bufa, sema)
    for k in range(nzcopy):
        r0 = r00 + k * CHUNK
        pltpu.make_async_copy(acc.at[pl.ds(r0, CHUNK)], obufs[k % 2],
                              osems[k % 2]).wait()
        if k + 1 < nzcopy:
            pltpu.async_copy(acc.at[pl.ds(r0 + CHUNK, CHUNK)],
                             obufs[(k + 1) % 2], osems[(k + 1) % 2])
        pltpu.sync_copy(obufs[k % 2], out.at[cid, pl.ds(r0, CHUNK)])


def _tc_mm_body(x_ref, w_ref, h_ref):
    h_ref[...] = jnp.dot(x_ref[...], w_ref[...],
                         preferred_element_type=jnp.float32,
                         precision=lax.Precision.HIGHEST)


def _tc_scale_body(h_ref, d0_ref, d1_ref, hp_ref, dinv_ref):
    deg = d0_ref[...] + d1_ref[...] + 1.0
    dinv = lax.rsqrt(deg)
    hp_ref[...] = h_ref[...] * dinv
    dinv_ref[...] = dinv


def _tc_mid_body(acc0_ref, acc1_ref, hp_ref, dinv_ref, b_ref, w_ref, out_ref):
    dinv = dinv_ref[...]
    z = (acc0_ref[0] + acc1_ref[0] + hp_ref[...]) * dinv + b_ref[...]
    z = jnp.maximum(z, 0.0)
    out_ref[...] = jnp.dot(
        z, w_ref[...], preferred_element_type=jnp.float32,
        precision=lax.Precision.HIGHEST) * dinv


def _tc_last_body(acc0_ref, acc1_ref, hp_ref, dinv_ref, b_ref, batch_ref,
                  fcw_ref, fcb_ref, out_ref, sums_ref, cnts_ref):
    i = pl.program_id(0)
    z = (acc0_ref[0] + acc1_ref[0] + hp_ref[...]) * dinv_ref[...] + b_ref[...]
    z = jnp.maximum(z, 0.0)
    gid = lax.broadcasted_iota(jnp.int32, (RB, G), 1)
    onehot = (batch_ref[...] == gid).astype(jnp.float32)
    ps = lax.dot_general(onehot, z, (((0,), (0,)), ((), ())),
                         preferred_element_type=jnp.float32,
        precision=lax.Precision.HIGHEST)
    pc = lax.dot_general(onehot, jnp.ones_like(z), (((0,), (0,)), ((), ())),
                         preferred_element_type=jnp.float32,
        precision=lax.Precision.HIGHEST)

    @pl.when(i == 0)
    def _():
        sums_ref[...] = ps
        cnts_ref[...] = pc

    @pl.when(i > 0)
    def _():
        sums_ref[...] += ps
        cnts_ref[...] += pc

    @pl.when(i == NRB - 1)
    def _():
        g = sums_ref[...] / jnp.maximum(cnts_ref[...], 1.0)
        out_ref[...] = jnp.dot(
            g, fcw_ref[...], preferred_element_type=jnp.float32,
        precision=lax.Precision.HIGHEST) + fcb_ref[...]


_row_spec = pl.BlockSpec((RB, D), lambda i: (i, 0))
_col_spec = pl.BlockSpec((RB, 1), lambda i: (i, 0))
_full_w = pl.BlockSpec((D, D), lambda i: (0, 0))
_full_b = pl.BlockSpec((1, D), lambda i: (0, 0))
_acc0_spec = pl.BlockSpec((1, RB, D), lambda i: (0, i, 0))
_acc1_spec = pl.BlockSpec((1, RB, D), lambda i: (1, i, 0))

_tc_mm = pl.pallas_call(
    _tc_mm_body,
    grid=(NRB,),
    in_specs=[_row_spec, _full_w],
    out_specs=_row_spec,
    out_shape=jax.ShapeDtypeStruct((N, D), jnp.float32),
)

_tc_scale = pl.pallas_call(
    _tc_scale_body,
    grid=(NRB,),
    in_specs=[_row_spec, _col_spec, _col_spec],
    out_specs=[_row_spec, _col_spec],
    out_shape=[jax.ShapeDtypeStruct((N, D), jnp.float32),
               jax.ShapeDtypeStruct((N, 1), jnp.float32)],
)

_tc_mid = pl.pallas_call(
    _tc_mid_body,
    grid=(NRB,),
    in_specs=[_acc0_spec, _acc1_spec, _row_spec, _col_spec, _full_b, _full_w],
    out_specs=_row_spec,
    out_shape=jax.ShapeDtypeStruct((N, D), jnp.float32),
)

_tc_last = pl.pallas_call(
    _tc_last_body,
    grid=(NRB,),
    in_specs=[_acc0_spec, _acc1_spec, _row_spec, _col_spec, _full_b,
              pl.BlockSpec((RB, 1), lambda i: (i, 0)),
              pl.BlockSpec((D, 1), lambda i: (0, 0)),
              pl.BlockSpec((1, 1), lambda i: (0, 0))],
    out_specs=pl.BlockSpec((G, 1), lambda i: (0, 0)),
    out_shape=jax.ShapeDtypeStruct((G, 1), jnp.float32),
    scratch_shapes=[pltpu.VMEM((G, D), jnp.float32),
                    pltpu.VMEM((G, D), jnp.float32)],
)


def kernel(x, edge_index, batch, W1, b1, W2, b2, fcW, fcb):
    src = edge_index[0].astype(jnp.int32)
    dst = edge_index[1].astype(jnp.int32)
    src4 = jnp.concatenate([src, _PAD_SRC]).reshape(NW, NSUPER, SB, CHUNK)
    dst4 = jnp.concatenate([dst, _PAD_DST]).reshape(NW, NSUPER, SB, CHUNK)

    ones_h = jnp.ones((CHUNK,), jnp.float32)
    zeros1 = jnp.zeros((NPAD // NS,), jnp.float32)
    zrows = jnp.zeros((CHUNK, D), jnp.float32)

    deg_parts = _sc_degree(dst4, ones_h, zeros1)
    h1 = _tc_mm(x, W1)
    d0 = deg_parts[0, :N].reshape(N, 1)
    d1 = deg_parts[1, :N].reshape(N, 1)
    h1p, dinv = _tc_scale(h1, d0, d1)

    acc1 = _sc_aggregate(h1p, src4, dst4, zrows)
    h2p = _tc_mid(acc1, acc1, h1p, dinv, b1.reshape(1, D), W2)

    acc2 = _sc_aggregate(h2p, src4, dst4, zrows)
    out = _tc_last(acc2, acc2, h2p, dinv, b2.reshape(1, D),
                   batch.astype(jnp.int32).reshape(N, 1), fcW,
                   fcb.reshape(1, 1))
    return out
